# Initial kernel scaffold; baseline (speedup 1.0000x reference)
import jax
import jax.numpy as jnp
from jax.experimental import pallas as pl


def kernel(x, w_router, b_router, w_c_fc, b_c_fc, w_gate, b_gate, w_c_proj, b_c_proj):
    raise NotImplementedError("write your pallas kernel here")


# speedup vs baseline: 5.6197x; 5.6197x over previous
import functools

import jax
import jax.numpy as jnp
from jax import lax
from jax.experimental import pallas as pl
from jax.experimental.pallas import tpu as pltpu
from jax.experimental.pallas import tpu_sc as plsc

BLK_T = 512
BM = 256

_SC_NC = 2
_SC_NS = 16
_SC_NW = _SC_NC * _SC_NS


def _router_body(x_ref, wr_ref, br_ref, eid_ref, counts_ref):
    i = pl.program_id(0)
    xb = x_ref[...]
    logits = jnp.dot(xb, wr_ref[...], preferred_element_type=jnp.float32)
    logits = logits + br_ref[...][None, :]
    n, e = logits.shape
    maxv = jnp.max(logits, axis=1, keepdims=True)
    lane = lax.broadcasted_iota(jnp.int32, (n, e), 1)
    eid = jnp.min(jnp.where(logits == maxv, lane, e), axis=1)
    eid_ref[0, 0, :] = eid
    oh = (lane == eid[:, None]).astype(jnp.float32)

    @pl.when(i == 0)
    def _():
        counts_ref[...] = jnp.zeros_like(counts_ref)

    counts_ref[...] += jnp.sum(oh, axis=0, keepdims=True)


def _plan_body(eid_ref, counts_ref, dest_ref, be_ref, running_ref):
    i = pl.program_id(0)

    @pl.when(i == 0)
    def _():
        running_ref[...] = jnp.zeros_like(running_ref)

    counts = counts_ref[...]
    e = counts.shape[1]
    pc = jnp.ceil(counts / BM) * BM
    tri = (lax.broadcasted_iota(jnp.int32, (e, e), 0)
           < lax.broadcasted_iota(jnp.int32, (e, e), 1)).astype(jnp.float32)
    offs = jnp.dot(pc, tri, preferred_element_type=jnp.float32)
    ends = offs + pc

    eid = eid_ref[0, 0, :]
    n = eid.shape[0]
    lane = lax.broadcasted_iota(jnp.int32, (n, e), 1)
    oh = (lane == eid[:, None]).astype(jnp.float32)
    stri = (lax.broadcasted_iota(jnp.int32, (n, n), 1)
            < lax.broadcasted_iota(jnp.int32, (n, n), 0)).astype(jnp.float32)
    earlier = jnp.dot(stri, oh, preferred_element_type=jnp.float32)
    base = offs + running_ref[...]
    destf = jnp.sum((base + earlier) * oh, axis=1)
    dest_ref[0, 0, :] = destf.astype(jnp.int32)
    running_ref[...] += jnp.sum(oh, axis=0, keepdims=True)

    @pl.when(i == 0)
    def _():
        jb = lax.broadcasted_iota(jnp.int32, (1, 128), 1).astype(jnp.float32) * BM
        lane8 = lax.broadcasted_iota(jnp.int32, (1, e), 1)
        acc = jnp.zeros((1, 128), jnp.float32)
        for ei in range(e):
            end_e = jnp.sum(jnp.where(lane8 == ei, ends, 0.0))
            acc += (jb >= end_e).astype(jnp.float32)
        be_ref[...] = jnp.minimum(acc, float(e - 1)).astype(jnp.int32)


def _gmm_body(be_ref, xs_ref, wfc_ref, bfc_ref, wg_ref, bg_ref, wp_ref,
              bp_ref, out_ref):
    del be_ref
    xb = xs_ref[...]
    h = jnp.dot(xb, wfc_ref[0], preferred_element_type=jnp.float32)
    h = h + bfc_ref[0]
    g = jnp.dot(xb, wg_ref[0], preferred_element_type=jnp.float32)
    g = g + bg_ref[0]
    g = g * (1.0 / (1.0 + jnp.exp(-g)))
    o = jnp.dot(h * g, wp_ref[0], preferred_element_type=jnp.float32)
    out_ref[...] = o + bp_ref[0]


def _sc_scatter(xf, dest, npad):
    tok, c = xf.shape
    per_w = tok // _SC_NW
    ch = 128
    n_ch = per_w // ch
    mesh = plsc.VectorSubcoreMesh(core_axis_name="c", subcore_axis_name="s")

    @functools.partial(
        pl.kernel, mesh=mesh,
        out_type=jax.ShapeDtypeStruct((npad, c), jnp.float32),
        scratch_types=[
            pltpu.VMEM((ch,), jnp.int32),
            pltpu.VMEM((ch, c), jnp.float32),
            pltpu.SemaphoreType.DMA,
        ],
    )
    def k(x_hbm, dest_hbm, out_hbm, idx_v, rows_v, sem):
        wid = lax.axis_index("s") * _SC_NC + lax.axis_index("c")
        base = wid * per_w
        for j in range(n_ch):
            o = base + j * ch
            pltpu.sync_copy(dest_hbm.at[pl.ds(o, ch)], idx_v)
            pltpu.sync_copy(x_hbm.at[pl.ds(o, ch)], rows_v)
            pltpu.async_copy(rows_v, out_hbm.at[idx_v], sem).wait()

    return k(xf, dest)


def _sc_gather(ys, dest, tok):
    _, c = ys.shape
    per_w = tok // _SC_NW
    ch = 128
    n_ch = per_w // ch
    mesh = plsc.VectorSubcoreMesh(core_axis_name="c", subcore_axis_name="s")

    @functools.partial(
        pl.kernel, mesh=mesh,
        out_type=jax.ShapeDtypeStruct((tok, c), jnp.float32),
        scratch_types=[
            pltpu.VMEM((ch,), jnp.int32),
            pltpu.VMEM((ch, c), jnp.float32),
            pltpu.SemaphoreType.DMA,
        ],
    )
    def k(y_hbm, dest_hbm, out_hbm, idx_v, rows_v, sem):
        wid = lax.axis_index("s") * _SC_NC + lax.axis_index("c")
        base = wid * per_w
        for j in range(n_ch):
            o = base + j * ch
            pltpu.sync_copy(dest_hbm.at[pl.ds(o, ch)], idx_v)
            pltpu.async_copy(y_hbm.at[idx_v], rows_v, sem).wait()
            pltpu.sync_copy(rows_v, out_hbm.at[pl.ds(o, ch)])

    return k(ys, dest)


def kernel(x, w_router, b_router, w_c_fc, b_c_fc, w_gate, b_gate, w_c_proj,
           b_c_proj):
    b, t, c = x.shape
    e, _, h = w_c_fc.shape
    tok = b * t
    n_rblk = tok // BLK_T
    nblk = tok // BM + e
    npad = nblk * BM
    xf = x.reshape(tok, c)

    eid3, counts = pl.pallas_call(
        _router_body,
        grid=(n_rblk,),
        in_specs=[
            pl.BlockSpec((BLK_T, c), lambda i: (i, 0)),
            pl.BlockSpec((c, e), lambda i: (0, 0)),
            pl.BlockSpec((e,), lambda i: (0,)),
        ],
        out_specs=[
            pl.BlockSpec((1, 1, BLK_T), lambda i: (i, 0, 0)),
            pl.BlockSpec((1, e), lambda i: (0, 0)),
        ],
        out_shape=[
            jax.ShapeDtypeStruct((n_rblk, 1, BLK_T), jnp.int32),
            jax.ShapeDtypeStruct((1, e), jnp.float32),
        ],
    )(xf, w_router, b_router)

    dest3, be = pl.pallas_call(
        _plan_body,
        grid=(n_rblk,),
        in_specs=[
            pl.BlockSpec((1, 1, BLK_T), lambda i: (i, 0, 0)),
            pl.BlockSpec((1, e), lambda i: (0, 0)),
        ],
        out_specs=[
            pl.BlockSpec((1, 1, BLK_T), lambda i: (i, 0, 0)),
            pl.BlockSpec((1, 128), lambda i: (0, 0)),
        ],
        out_shape=[
            jax.ShapeDtypeStruct((n_rblk, 1, BLK_T), jnp.int32),
            jax.ShapeDtypeStruct((1, 128), jnp.int32),
        ],
        scratch_shapes=[pltpu.VMEM((1, e), jnp.float32)],
    )(eid3, counts)

    dest = dest3.reshape(tok)
    be_list = be.reshape(128)[:nblk]

    xs = _sc_scatter(xf, dest, npad)

    ys = pl.pallas_call(
        _gmm_body,
        grid_spec=pltpu.PrefetchScalarGridSpec(
            num_scalar_prefetch=1,
            grid=(nblk,),
            in_specs=[
                pl.BlockSpec((BM, c), lambda i, be: (i, 0)),
                pl.BlockSpec((1, c, h), lambda i, be: (be[i], 0, 0)),
                pl.BlockSpec((1, 1, h), lambda i, be: (be[i], 0, 0)),
                pl.BlockSpec((1, c, h), lambda i, be: (be[i], 0, 0)),
                pl.BlockSpec((1, 1, h), lambda i, be: (be[i], 0, 0)),
                pl.BlockSpec((1, h, c), lambda i, be: (be[i], 0, 0)),
                pl.BlockSpec((1, 1, c), lambda i, be: (be[i], 0, 0)),
            ],
            out_specs=pl.BlockSpec((BM, c), lambda i, be: (i, 0)),
        ),
        out_shape=jax.ShapeDtypeStruct((npad, c), jnp.float32),
    )(be_list, xs, w_c_fc, b_c_fc, w_gate, b_gate, w_c_proj, b_c_proj)

    yf = _sc_gather(ys, dest, tok)
    return yf.reshape(b, t, c)
